# baseline (speedup 1.0000x reference)
import functools

import jax
import jax.numpy as jnp
from jax import lax
from jax.experimental import pallas as pl
from jax.experimental.pallas import tpu as pltpu
from jax.experimental.pallas import tpu_sc as plsc

DE = 64
TOKEN_SIZE = 100000
BATCH = 4
SEQ = 2048

_info = plsc.get_sparse_core_info()
NC, NS = _info.num_cores, _info.num_subcores
NW = NC * NS
NPASS = DE // NW
GRP = SEQ // 16


def _embed_body(idx_hbm, tok_hbm, pos_hbm, out_hbm,
                idx_v, row_v, pos_v, out0, out1, sem, osem):
    wid = lax.axis_index("s") * NC + lax.axis_index("c")
    dims = [p * NW + wid for p in range(NPASS)]
    outs = (out0, out1)

    row_cp = pltpu.async_copy(tok_hbm.at[dims[0]], row_v, sem)
    pltpu.sync_copy(idx_hbm, idx_v)
    for p in range(NPASS):
        pltpu.sync_copy(pos_hbm.at[dims[p]], pos_v.at[p])

    out_cps = []
    for p in range(NPASS):
        row_cp.wait()
        out_v = outs[p]

        @plsc.parallel_loop(0, GRP, unroll=8)
        def gather_add(g, out_v=out_v, p=p):
            sl = pl.ds(g * 16, 16)
            csl = pl.ds((g % 8) * 16, 16)
            pv = pos_v[p, sl]
            for b in range(BATCH):
                ids = idx_v[(g // 8) * BATCH + b, csl]
                out_v[b, sl] = plsc.load_gather(row_v, [ids]) + pv

        if p + 1 < NPASS:
            row_cp = pltpu.async_copy(tok_hbm.at[dims[p + 1]], row_v, sem)
        for b in range(BATCH):
            out_cps.append(
                pltpu.async_copy(out_v.at[b], out_hbm.at[b, dims[p]], osem))

    for cp in out_cps:
        cp.wait()


@functools.partial(
    pl.kernel,
    mesh=plsc.VectorSubcoreMesh(core_axis_name="c", subcore_axis_name="s"),
    out_type=jax.ShapeDtypeStruct((BATCH, DE, SEQ), jnp.float32),
    compiler_params=pltpu.CompilerParams(needs_layout_passes=False),
    scratch_types=[
        pltpu.VMEM((BATCH * SEQ // 128, 128), jnp.int32),
        pltpu.VMEM((TOKEN_SIZE,), jnp.float32),
        pltpu.VMEM((NPASS, SEQ), jnp.float32),
        pltpu.VMEM((BATCH, SEQ), jnp.float32),
        pltpu.VMEM((BATCH, SEQ), jnp.float32),
        pltpu.SemaphoreType.DMA,
        pltpu.SemaphoreType.DMA,
    ],
)
def _embed(idx_hbm, tok_hbm, pos_hbm, out_hbm,
           idx_v, row_v, pos_v, out0, out1, sem, osem):
    _embed_body(idx_hbm, tok_hbm, pos_hbm, out_hbm,
                idx_v, row_v, pos_v, out0, out1, sem, osem)


def kernel(inputs, token_table, pos_table):
    idx = inputs.astype(jnp.int32).reshape(BATCH, SEQ // 128, 128)
    idx = jnp.transpose(idx, (1, 0, 2)).reshape(BATCH * SEQ // 128, 128)
    out = _embed(idx, token_table.T, pos_table.T)
    return jnp.transpose(out, (0, 2, 1))


# speedup vs baseline: 1.0027x; 1.0027x over previous
import functools

import jax
import jax.numpy as jnp
from jax import lax
from jax.experimental import pallas as pl
from jax.experimental.pallas import tpu as pltpu
from jax.experimental.pallas import tpu_sc as plsc

DE = 64
TOKEN_SIZE = 100000
BATCH = 4
SEQ = 2048

_info = plsc.get_sparse_core_info()
NC, NS = _info.num_cores, _info.num_subcores
NW = NC * NS
NPASS = DE // NW
GRP = SEQ // 16


def _embed_body(idx_hbm, tok_hbm, pos_hbm, out_hbm,
                idx_v, row_v, pos_v, out0, out1, sem, osem):
    wid = lax.axis_index("s") * NC + lax.axis_index("c")
    dims = [p * NW + wid for p in range(NPASS)]
    outs = (out0, out1)

    row_cp = pltpu.async_copy(tok_hbm.at[dims[0]], row_v, sem)
    pltpu.sync_copy(idx_hbm, idx_v)
    for p in range(NPASS):
        pltpu.sync_copy(pos_hbm.at[dims[p]], pos_v.at[p])

    out_cps = []
    for p in range(NPASS):
        row_cp.wait()
        out_v = outs[p]

        @plsc.parallel_loop(0, GRP, unroll=4)
        def gather_add(g, out_v=out_v, p=p):
            sl = pl.ds(g * 16, 16)
            csl = pl.ds((g % 8) * 16, 16)
            pv = pos_v[p, sl]
            for b in range(BATCH):
                ids = idx_v[(g // 8) * BATCH + b, csl]
                out_v[b, sl] = plsc.load_gather(row_v, [ids]) + pv

        if p + 1 < NPASS:
            row_cp = pltpu.async_copy(tok_hbm.at[dims[p + 1]], row_v, sem)
        for b in range(BATCH):
            out_cps.append(
                pltpu.async_copy(out_v.at[b], out_hbm.at[b, dims[p]], osem))

    for cp in out_cps:
        cp.wait()


@functools.partial(
    pl.kernel,
    mesh=plsc.VectorSubcoreMesh(core_axis_name="c", subcore_axis_name="s"),
    out_type=jax.ShapeDtypeStruct((BATCH, DE, SEQ), jnp.float32),
    compiler_params=pltpu.CompilerParams(needs_layout_passes=False),
    scratch_types=[
        pltpu.VMEM((BATCH * SEQ // 128, 128), jnp.int32),
        pltpu.VMEM((TOKEN_SIZE,), jnp.float32),
        pltpu.VMEM((NPASS, SEQ), jnp.float32),
        pltpu.VMEM((BATCH, SEQ), jnp.float32),
        pltpu.VMEM((BATCH, SEQ), jnp.float32),
        pltpu.SemaphoreType.DMA,
        pltpu.SemaphoreType.DMA,
    ],
)
def _embed(idx_hbm, tok_hbm, pos_hbm, out_hbm,
           idx_v, row_v, pos_v, out0, out1, sem, osem):
    _embed_body(idx_hbm, tok_hbm, pos_hbm, out_hbm,
                idx_v, row_v, pos_v, out0, out1, sem, osem)


def kernel(inputs, token_table, pos_table):
    idx = inputs.astype(jnp.int32).reshape(BATCH, SEQ // 128, 128)
    idx = jnp.transpose(idx, (1, 0, 2)).reshape(BATCH * SEQ // 128, 128)
    out = _embed(idx, token_table.T, pos_table.T)
    return jnp.transpose(out, (0, 2, 1))
